# baseline (speedup 1.0000x reference)
import functools

import jax
import jax.numpy as jnp
from jax import lax
from jax.experimental import pallas as pl
from jax.experimental.pallas import tpu as pltpu
from jax.experimental.pallas import tpu_sc as plsc

C_IN = 100000
D_MODEL = 64
BATCH = 4096
HIST = 50

NC = 2
NS = 16
NW = NC * NS

B = BATCH * HIST
K = 128
B_PER_W = B // NW
STEPS = B_PER_W // K
NBUF = 5
LA = 3
INNER = NBUF

CH = 128


def _gather_body(x_hbm, table_hbm, out_hbm, idx_v, rows, gsems, osems, *, steps):
    wid = lax.axis_index("s") * NC + lax.axis_index("c")
    base = wid * steps * K
    pltpu.sync_copy(x_hbm.at[pl.ds(wid * steps, steps)], idx_v)

    def gather(j, b):
        return pltpu.make_async_copy(
            table_hbm.at[idx_v.at[j]], rows[b], gsems[b]
        )

    def writeback(j, b):
        return pltpu.make_async_copy(
            rows[b], out_hbm.at[pl.ds(base + j * K, K)], osems[b]
        )

    for j in range(LA):
        gather(j, j).start()

    @pl.loop(0, steps // INNER)
    def _outer(p):
        j0 = p * INNER
        for t in range(INNER):
            j = j0 + t
            b = t % NBUF
            nb = (t + LA) % NBUF
            if t >= NBUF - LA:
                writeback(j + LA - NBUF, nb).wait()
            else:
                @pl.when(p > 0)
                def _():
                    writeback(j + LA - NBUF, nb).wait()

            @pl.when(j + LA < steps)
            def _():
                gather(j + LA, nb).start()

            gather(j, b).wait()
            writeback(j, b).start()

    for t in range(NBUF - LA):
        jt = steps - (NBUF - LA) + t
        writeback(jt, jt % NBUF).wait()


CP = 102400
HALF2 = CP // 2
CHW = 1024


def _repack_body(a_ref, b_ref, o_ref):
    o_ref[:, :D_MODEL] = a_ref[...].T
    o_ref[:, D_MODEL:] = b_ref[...].T


def _format_body_alias(dest_ref, src_ref, out_ref):
    del dest_ref
    _format_core(src_ref, out_ref)


def _format_body(src_ref, out_ref):
    _format_core(src_ref, out_ref)


def _format_core(src_ref, out_ref):
    m = src_ref[...].reshape(CH, HIST * D_MODEL)
    out_ref[...] = m.T.reshape(HIST, D_MODEL, CH)


@jax.jit
def kernel(x, W):
    x_t = jnp.where(x < HALF2, 2 * x, 2 * (x - HALF2) + 1)
    x_flat = x_t.reshape(B // K, K)
    w_t = jnp.pad(W, ((0, CP - C_IN), (0, 0))).T
    w_pairs = pl.pallas_call(
        _repack_body,
        grid=(HALF2 // CHW,),
        in_specs=[
            pl.BlockSpec((D_MODEL, CHW), lambda i: (0, i)),
            pl.BlockSpec((D_MODEL, CHW), lambda i: (0, i + HALF2 // CHW)),
        ],
        out_specs=pl.BlockSpec((CHW, 2 * D_MODEL), lambda i: (i, 0)),
        out_shape=jax.ShapeDtypeStruct((HALF2, 2 * D_MODEL), jnp.float32),
    )(w_t, w_t)
    w_lin = w_pairs.reshape(CP, D_MODEL)
    mesh = plsc.VectorSubcoreMesh(core_axis_name="c", subcore_axis_name="s")

    hsteps = STEPS // 2
    hb = B // 2

    def sc_gather(x_half):
        return pl.kernel(
            functools.partial(_gather_body, steps=hsteps),
            out_type=jax.ShapeDtypeStruct((hb, D_MODEL), jnp.float32),
            mesh=mesh,
            scratch_types=[
                pltpu.VMEM((hsteps, K), jnp.int32),
                tuple(pltpu.VMEM((K, D_MODEL), jnp.float32) for _ in range(NBUF)),
                tuple(pltpu.SemaphoreType.DMA for _ in range(NBUF)),
                tuple(pltpu.SemaphoreType.DMA for _ in range(NBUF)),
            ],
            compiler_params=pltpu.CompilerParams(use_tc_tiling_on_sc=False),
        )(x_half, w_lin)

    flat_a = sc_gather(x_flat[: B // (2 * K)])
    flat_b = sc_gather(x_flat[B // (2 * K):])

    rows_per_ch = CH * HIST // 2
    half_grid = BATCH // (2 * CH)
    wide_a = flat_a.reshape(hb // 2, 2 * D_MODEL)
    wide_b = flat_b.reshape(hb // 2, 2 * D_MODEL)
    part = pl.pallas_call(
        _format_body,
        grid=(half_grid,),
        in_specs=[pl.BlockSpec((rows_per_ch, 2 * D_MODEL), lambda i: (i, 0))],
        out_specs=pl.BlockSpec((HIST, D_MODEL, CH), lambda i: (0, 0, i)),
        out_shape=jax.ShapeDtypeStruct((HIST, D_MODEL, BATCH), jnp.float32),
    )(wide_a)
    tbd = pl.pallas_call(
        _format_body_alias,
        grid=(half_grid,),
        in_specs=[
            pl.BlockSpec(memory_space=pl.ANY),
            pl.BlockSpec((rows_per_ch, 2 * D_MODEL), lambda i: (i, 0)),
        ],
        out_specs=pl.BlockSpec(
            (HIST, D_MODEL, CH), lambda i: (0, 0, i + half_grid)
        ),
        out_shape=jax.ShapeDtypeStruct((HIST, D_MODEL, BATCH), jnp.float32),
        input_output_aliases={0: 0},
    )(part, wide_b)
    return jnp.transpose(tbd, (2, 0, 1))


# speedup vs baseline: 1.0353x; 1.0353x over previous
import functools

import jax
import jax.numpy as jnp
from jax import lax
from jax.experimental import pallas as pl
from jax.experimental.pallas import tpu as pltpu
from jax.experimental.pallas import tpu_sc as plsc

C_IN = 100000
D_MODEL = 64
BATCH = 4096
HIST = 50

NC = 2
NS = 16
NW = NC * NS

B = BATCH * HIST
K = 128
B_PER_W = B // NW
STEPS = B_PER_W // K
NBUF = 5
LA = 3
INNER = NBUF

CH = 128


def _gather_body(x_hbm, table_hbm, out_hbm, idx_v, rows, gsems, osems, *, steps):
    wid = lax.axis_index("s") * NC + lax.axis_index("c")
    base = wid * steps * K
    pltpu.sync_copy(x_hbm.at[pl.ds(wid * steps, steps)], idx_v)

    def gather(j, b):
        return pltpu.make_async_copy(
            table_hbm.at[idx_v.at[j]], rows[b], gsems[b]
        )

    def writeback(j, b):
        return pltpu.make_async_copy(
            rows[b], out_hbm.at[pl.ds(base + j * K, K)], osems[b]
        )

    for j in range(LA):
        gather(j, j).start()

    @pl.loop(0, steps // INNER)
    def _outer(p):
        j0 = p * INNER
        for t in range(INNER):
            j = j0 + t
            b = t % NBUF
            nb = (t + LA) % NBUF
            if t >= NBUF - LA:
                writeback(j + LA - NBUF, nb).wait()
            else:
                @pl.when(p > 0)
                def _():
                    writeback(j + LA - NBUF, nb).wait()

            @pl.when(j + LA < steps)
            def _():
                gather(j + LA, nb).start()

            gather(j, b).wait()
            writeback(j, b).start()

    for t in range(NBUF - LA):
        jt = steps - (NBUF - LA) + t
        writeback(jt, jt % NBUF).wait()


CP = 102400
HALF2 = CP // 2
CHW = 1024


def _repack_body(a_ref, o_ref):
    i = pl.program_id(0)
    o_ref[:, :D_MODEL] = a_ref[:, pl.ds(i * CHW, CHW)].T
    o_ref[:, D_MODEL:] = a_ref[:, pl.ds(HALF2 + i * CHW, CHW)].T


def _format_body_alias(dest_ref, src_ref, out_ref):
    del dest_ref
    _format_core(src_ref, out_ref)


def _format_body(src_ref, out_ref):
    _format_core(src_ref, out_ref)


def _format_core(src_ref, out_ref):
    m = src_ref[...].reshape(CH, HIST * D_MODEL)
    out_ref[...] = m.T.reshape(HIST, D_MODEL, CH)


@jax.jit
def kernel(x, W):
    x_t = jnp.where(x < HALF2, 2 * x, 2 * (x - HALF2) + 1)
    x_flat = x_t.reshape(B // K, K)
    w_t = jnp.pad(W, ((0, CP - C_IN), (0, 0))).T
    w_pairs = pl.pallas_call(
        _repack_body,
        grid=(HALF2 // CHW,),
        in_specs=[pl.BlockSpec((D_MODEL, CP), lambda i: (0, 0))],
        out_specs=pl.BlockSpec((CHW, 2 * D_MODEL), lambda i: (i, 0)),
        out_shape=jax.ShapeDtypeStruct((HALF2, 2 * D_MODEL), jnp.float32),
    )(w_t)
    w_lin = w_pairs.reshape(CP, D_MODEL)
    mesh = plsc.VectorSubcoreMesh(core_axis_name="c", subcore_axis_name="s")

    hsteps = STEPS // 2
    hb = B // 2

    def sc_gather(x_half):
        return pl.kernel(
            functools.partial(_gather_body, steps=hsteps),
            out_type=jax.ShapeDtypeStruct((hb, D_MODEL), jnp.float32),
            mesh=mesh,
            scratch_types=[
                pltpu.VMEM((hsteps, K), jnp.int32),
                tuple(pltpu.VMEM((K, D_MODEL), jnp.float32) for _ in range(NBUF)),
                tuple(pltpu.SemaphoreType.DMA for _ in range(NBUF)),
                tuple(pltpu.SemaphoreType.DMA for _ in range(NBUF)),
            ],
            compiler_params=pltpu.CompilerParams(use_tc_tiling_on_sc=False),
        )(x_half, w_lin)

    flat_a = sc_gather(x_flat[: B // (2 * K)])
    flat_b = sc_gather(x_flat[B // (2 * K):])

    rows_per_ch = CH * HIST // 2
    half_grid = BATCH // (2 * CH)
    wide_a = flat_a.reshape(hb // 2, 2 * D_MODEL)
    wide_b = flat_b.reshape(hb // 2, 2 * D_MODEL)
    part = pl.pallas_call(
        _format_body,
        grid=(half_grid,),
        in_specs=[pl.BlockSpec((rows_per_ch, 2 * D_MODEL), lambda i: (i, 0))],
        out_specs=pl.BlockSpec((HIST, D_MODEL, CH), lambda i: (0, 0, i)),
        out_shape=jax.ShapeDtypeStruct((HIST, D_MODEL, BATCH), jnp.float32),
    )(wide_a)
    tbd = pl.pallas_call(
        _format_body_alias,
        grid=(half_grid,),
        in_specs=[
            pl.BlockSpec(memory_space=pl.ANY),
            pl.BlockSpec((rows_per_ch, 2 * D_MODEL), lambda i: (i, 0)),
        ],
        out_specs=pl.BlockSpec(
            (HIST, D_MODEL, CH), lambda i: (0, 0, i + half_grid)
        ),
        out_shape=jax.ShapeDtypeStruct((HIST, D_MODEL, BATCH), jnp.float32),
        input_output_aliases={0: 0},
    )(part, wide_b)
    return jnp.transpose(tbd, (2, 0, 1))


# speedup vs baseline: 1.0929x; 1.0557x over previous
import functools

import jax
import jax.numpy as jnp
from jax import lax
from jax.experimental import pallas as pl
from jax.experimental.pallas import tpu as pltpu
from jax.experimental.pallas import tpu_sc as plsc

C_IN = 100000
D_MODEL = 64
BATCH = 4096
HIST = 50

NC = 2
NS = 16
NW = NC * NS

B = BATCH * HIST
K = 128
B_PER_W = B // NW
STEPS = B_PER_W // K
NBUF = 5
LA = 3
INNER = NBUF

CH = 128


def _gather_body(x_hbm, table_hbm, out_hbm, idx_v, rows, gsems, osems, *, steps):
    wid = lax.axis_index("s") * NC + lax.axis_index("c")
    base = wid * steps * K
    pltpu.sync_copy(x_hbm.at[pl.ds(wid * steps, steps)], idx_v)

    def gather(j, b):
        return pltpu.make_async_copy(
            table_hbm.at[idx_v.at[j]], rows[b], gsems[b]
        )

    def writeback(j, b):
        return pltpu.make_async_copy(
            rows[b], out_hbm.at[pl.ds(base + j * K, K)], osems[b]
        )

    for j in range(LA):
        gather(j, j).start()

    @pl.loop(0, steps // INNER)
    def _outer(p):
        j0 = p * INNER
        for t in range(INNER):
            j = j0 + t
            b = t % NBUF
            nb = (t + LA) % NBUF
            if t >= NBUF - LA:
                writeback(j + LA - NBUF, nb).wait()
            else:
                @pl.when(p > 0)
                def _():
                    writeback(j + LA - NBUF, nb).wait()

            @pl.when(j + LA < steps)
            def _():
                gather(j + LA, nb).start()

            gather(j, b).wait()
            writeback(j, b).start()

    for t in range(NBUF - LA):
        jt = steps - (NBUF - LA) + t
        writeback(jt, jt % NBUF).wait()


CP = 102400
HALF2 = CP // 2
CHW = 2048


def _repack_body(a_ref, o_ref):
    i = pl.program_id(0)
    o_ref[:, :D_MODEL] = a_ref[:, pl.ds(i * CHW, CHW)].T
    o_ref[:, D_MODEL:] = a_ref[:, pl.ds(HALF2 + i * CHW, CHW)].T


def _format_body_alias(dest_ref, src_ref, out_ref):
    del dest_ref
    _format_core(src_ref, out_ref)


def _format_body(src_ref, out_ref):
    _format_core(src_ref, out_ref)


def _format_core(src_ref, out_ref):
    m = src_ref[...].reshape(CH, HIST * D_MODEL)
    out_ref[...] = m.T.reshape(HIST, D_MODEL, CH)


@jax.jit
def kernel(x, W):
    x_t = jnp.where(x < HALF2, 2 * x, 2 * (x - HALF2) + 1)
    x_flat = x_t.reshape(B // K, K)
    w_t = jnp.pad(W, ((0, CP - C_IN), (0, 0))).T
    w_pairs = pl.pallas_call(
        _repack_body,
        grid=(HALF2 // CHW,),
        in_specs=[pl.BlockSpec((D_MODEL, CP), lambda i: (0, 0))],
        out_specs=pl.BlockSpec((CHW, 2 * D_MODEL), lambda i: (i, 0)),
        out_shape=jax.ShapeDtypeStruct((HALF2, 2 * D_MODEL), jnp.float32),
    )(w_t)
    w_lin = w_pairs.reshape(CP, D_MODEL)
    mesh = plsc.VectorSubcoreMesh(core_axis_name="c", subcore_axis_name="s")

    hsteps = STEPS // 2
    hb = B // 2

    def sc_gather(x_half):
        return pl.kernel(
            functools.partial(_gather_body, steps=hsteps),
            out_type=jax.ShapeDtypeStruct((hb, D_MODEL), jnp.float32),
            mesh=mesh,
            scratch_types=[
                pltpu.VMEM((hsteps, K), jnp.int32),
                tuple(pltpu.VMEM((K, D_MODEL), jnp.float32) for _ in range(NBUF)),
                tuple(pltpu.SemaphoreType.DMA for _ in range(NBUF)),
                tuple(pltpu.SemaphoreType.DMA for _ in range(NBUF)),
            ],
            compiler_params=pltpu.CompilerParams(use_tc_tiling_on_sc=False),
        )(x_half, w_lin)

    flat_a = sc_gather(x_flat[: B // (2 * K)])
    flat_b = sc_gather(x_flat[B // (2 * K):])

    rows_per_ch = CH * HIST // 2
    half_grid = BATCH // (2 * CH)
    wide_a = flat_a.reshape(hb // 2, 2 * D_MODEL)
    wide_b = flat_b.reshape(hb // 2, 2 * D_MODEL)
    part = pl.pallas_call(
        _format_body,
        grid=(half_grid,),
        in_specs=[pl.BlockSpec((rows_per_ch, 2 * D_MODEL), lambda i: (i, 0))],
        out_specs=pl.BlockSpec((HIST, D_MODEL, CH), lambda i: (0, 0, i)),
        out_shape=jax.ShapeDtypeStruct((HIST, D_MODEL, BATCH), jnp.float32),
    )(wide_a)
    tbd = pl.pallas_call(
        _format_body_alias,
        grid=(half_grid,),
        in_specs=[
            pl.BlockSpec(memory_space=pl.ANY),
            pl.BlockSpec((rows_per_ch, 2 * D_MODEL), lambda i: (i, 0)),
        ],
        out_specs=pl.BlockSpec(
            (HIST, D_MODEL, CH), lambda i: (0, 0, i + half_grid)
        ),
        out_shape=jax.ShapeDtypeStruct((HIST, D_MODEL, BATCH), jnp.float32),
        input_output_aliases={0: 0},
    )(part, wide_b)
    return jnp.transpose(tbd, (2, 0, 1))


# speedup vs baseline: 1.1046x; 1.0107x over previous
import functools

import jax
import jax.numpy as jnp
from jax import lax
from jax.experimental import pallas as pl
from jax.experimental.pallas import tpu as pltpu
from jax.experimental.pallas import tpu_sc as plsc

C_IN = 100000
D_MODEL = 64
BATCH = 4096
HIST = 50

NC = 2
NS = 16
NW = NC * NS

B = BATCH * HIST
K = 128
B_PER_W = B // NW
STEPS = B_PER_W // K
NBUF = 5
LA = 3
INNER = NBUF

CH = 128


def _gather_body(x_hbm, table_hbm, out_hbm, idx_v, rows, gsems, osems, *, steps):
    wid = lax.axis_index("s") * NC + lax.axis_index("c")
    base = wid * steps * K
    pltpu.sync_copy(x_hbm.at[pl.ds(wid * steps, steps)], idx_v)

    def gather(j, b):
        return pltpu.make_async_copy(
            table_hbm.at[idx_v.at[j]], rows[b], gsems[b]
        )

    def writeback(j, b):
        return pltpu.make_async_copy(
            rows[b], out_hbm.at[pl.ds(base + j * K, K)], osems[b]
        )

    for j in range(LA):
        gather(j, j).start()

    @pl.loop(0, steps // INNER)
    def _outer(p):
        j0 = p * INNER
        for t in range(INNER):
            j = j0 + t
            b = t % NBUF
            nb = (t + LA) % NBUF
            if t >= NBUF - LA:
                writeback(j + LA - NBUF, nb).wait()
            else:
                @pl.when(p > 0)
                def _():
                    writeback(j + LA - NBUF, nb).wait()

            @pl.when(j + LA < steps)
            def _():
                gather(j + LA, nb).start()

            gather(j, b).wait()
            writeback(j, b).start()

    for t in range(NBUF - LA):
        jt = steps - (NBUF - LA) + t
        writeback(jt, jt % NBUF).wait()


CP = 102400
HALF2 = CP // 2
CHW = 3200


def _repack_body(a_ref, o_ref):
    i = pl.program_id(0)
    o_ref[:, :D_MODEL] = a_ref[:, pl.ds(i * CHW, CHW)].T
    o_ref[:, D_MODEL:] = a_ref[:, pl.ds(HALF2 + i * CHW, CHW)].T


def _format_body_alias(dest_ref, src_ref, out_ref):
    del dest_ref
    _format_core(src_ref, out_ref)


def _format_body(src_ref, out_ref):
    _format_core(src_ref, out_ref)


def _format_core(src_ref, out_ref):
    m = src_ref[...].reshape(CH, HIST * D_MODEL)
    out_ref[...] = m.T.reshape(HIST, D_MODEL, CH)


@jax.jit
def kernel(x, W):
    x_t = jnp.where(x < HALF2, 2 * x, 2 * (x - HALF2) + 1)
    x_flat = x_t.reshape(B // K, K)
    w_t = jnp.pad(W, ((0, CP - C_IN), (0, 0))).T
    w_pairs = pl.pallas_call(
        _repack_body,
        grid=(HALF2 // CHW,),
        in_specs=[pl.BlockSpec((D_MODEL, CP), lambda i: (0, 0))],
        out_specs=pl.BlockSpec((CHW, 2 * D_MODEL), lambda i: (i, 0)),
        out_shape=jax.ShapeDtypeStruct((HALF2, 2 * D_MODEL), jnp.float32),
    )(w_t)
    w_lin = w_pairs.reshape(CP, D_MODEL)
    mesh = plsc.VectorSubcoreMesh(core_axis_name="c", subcore_axis_name="s")

    hsteps = STEPS // 2
    hb = B // 2

    def sc_gather(x_half):
        return pl.kernel(
            functools.partial(_gather_body, steps=hsteps),
            out_type=jax.ShapeDtypeStruct((hb, D_MODEL), jnp.float32),
            mesh=mesh,
            scratch_types=[
                pltpu.VMEM((hsteps, K), jnp.int32),
                tuple(pltpu.VMEM((K, D_MODEL), jnp.float32) for _ in range(NBUF)),
                tuple(pltpu.SemaphoreType.DMA for _ in range(NBUF)),
                tuple(pltpu.SemaphoreType.DMA for _ in range(NBUF)),
            ],
            compiler_params=pltpu.CompilerParams(use_tc_tiling_on_sc=False),
        )(x_half, w_lin)

    flat_a = sc_gather(x_flat[: B // (2 * K)])
    flat_b = sc_gather(x_flat[B // (2 * K):])

    rows_per_ch = CH * HIST // 2
    half_grid = BATCH // (2 * CH)
    wide_a = flat_a.reshape(hb // 2, 2 * D_MODEL)
    wide_b = flat_b.reshape(hb // 2, 2 * D_MODEL)
    part = pl.pallas_call(
        _format_body,
        grid=(half_grid,),
        in_specs=[pl.BlockSpec((rows_per_ch, 2 * D_MODEL), lambda i: (i, 0))],
        out_specs=pl.BlockSpec((HIST, D_MODEL, CH), lambda i: (0, 0, i)),
        out_shape=jax.ShapeDtypeStruct((HIST, D_MODEL, BATCH), jnp.float32),
    )(wide_a)
    tbd = pl.pallas_call(
        _format_body_alias,
        grid=(half_grid,),
        in_specs=[
            pl.BlockSpec(memory_space=pl.ANY),
            pl.BlockSpec((rows_per_ch, 2 * D_MODEL), lambda i: (i, 0)),
        ],
        out_specs=pl.BlockSpec(
            (HIST, D_MODEL, CH), lambda i: (0, 0, i + half_grid)
        ),
        out_shape=jax.ShapeDtypeStruct((HIST, D_MODEL, BATCH), jnp.float32),
        input_output_aliases={0: 0},
    )(part, wide_b)
    return jnp.transpose(tbd, (2, 0, 1))


# speedup vs baseline: 1.1505x; 1.0415x over previous
import functools

import jax
import jax.numpy as jnp
from jax import lax
from jax.experimental import pallas as pl
from jax.experimental.pallas import tpu as pltpu
from jax.experimental.pallas import tpu_sc as plsc

C_IN = 100000
D_MODEL = 64
BATCH = 4096
HIST = 50

NC = 2
NS = 16
NW = NC * NS

B = BATCH * HIST
K = 128
B_PER_W = B // NW
STEPS = B_PER_W // K
NBUF = 5
LA = 3
INNER = NBUF

CH = 256


def _gather_body(x_hbm, table_hbm, out_hbm, idx_v, rows, gsems, osems, *, steps):
    wid = lax.axis_index("s") * NC + lax.axis_index("c")
    base = wid * steps * K
    pltpu.sync_copy(x_hbm.at[pl.ds(wid * steps, steps)], idx_v)

    def gather(j, b):
        return pltpu.make_async_copy(
            table_hbm.at[idx_v.at[j]], rows[b], gsems[b]
        )

    def writeback(j, b):
        return pltpu.make_async_copy(
            rows[b], out_hbm.at[pl.ds(base + j * K, K)], osems[b]
        )

    for j in range(LA):
        gather(j, j).start()

    @pl.loop(0, steps // INNER)
    def _outer(p):
        j0 = p * INNER
        for t in range(INNER):
            j = j0 + t
            b = t % NBUF
            nb = (t + LA) % NBUF
            if t >= NBUF - LA:
                writeback(j + LA - NBUF, nb).wait()
            else:
                @pl.when(p > 0)
                def _():
                    writeback(j + LA - NBUF, nb).wait()

            @pl.when(j + LA < steps)
            def _():
                gather(j + LA, nb).start()

            gather(j, b).wait()
            writeback(j, b).start()

    for t in range(NBUF - LA):
        jt = steps - (NBUF - LA) + t
        writeback(jt, jt % NBUF).wait()


CP = 102400
HALF2 = CP // 2
CHW = 3200


def _repack_body(a_ref, o_ref):
    i = pl.program_id(0)
    o_ref[:, :D_MODEL] = a_ref[:, pl.ds(i * CHW, CHW)].T
    o_ref[:, D_MODEL:] = a_ref[:, pl.ds(HALF2 + i * CHW, CHW)].T


def _format_body_alias(dest_ref, src_ref, out_ref):
    del dest_ref
    _format_core(src_ref, out_ref)


def _format_body(src_ref, out_ref):
    _format_core(src_ref, out_ref)


def _format_core(src_ref, out_ref):
    m = src_ref[...].reshape(CH, HIST * D_MODEL)
    out_ref[...] = m.T.reshape(HIST, D_MODEL, CH)


@jax.jit
def kernel(x, W):
    x_t = jnp.where(x < HALF2, 2 * x, 2 * (x - HALF2) + 1)
    x_flat = x_t.reshape(B // K, K)
    w_t = jnp.pad(W, ((0, CP - C_IN), (0, 0))).T
    w_pairs = pl.pallas_call(
        _repack_body,
        grid=(HALF2 // CHW,),
        in_specs=[pl.BlockSpec((D_MODEL, CP), lambda i: (0, 0))],
        out_specs=pl.BlockSpec((CHW, 2 * D_MODEL), lambda i: (i, 0)),
        out_shape=jax.ShapeDtypeStruct((HALF2, 2 * D_MODEL), jnp.float32),
    )(w_t)
    w_lin = w_pairs.reshape(CP, D_MODEL)
    mesh = plsc.VectorSubcoreMesh(core_axis_name="c", subcore_axis_name="s")

    hsteps = STEPS // 2
    hb = B // 2

    def sc_gather(x_half):
        return pl.kernel(
            functools.partial(_gather_body, steps=hsteps),
            out_type=jax.ShapeDtypeStruct((hb, D_MODEL), jnp.float32),
            mesh=mesh,
            scratch_types=[
                pltpu.VMEM((hsteps, K), jnp.int32),
                tuple(pltpu.VMEM((K, D_MODEL), jnp.float32) for _ in range(NBUF)),
                tuple(pltpu.SemaphoreType.DMA for _ in range(NBUF)),
                tuple(pltpu.SemaphoreType.DMA for _ in range(NBUF)),
            ],
            compiler_params=pltpu.CompilerParams(use_tc_tiling_on_sc=False),
        )(x_half, w_lin)

    flat_a = sc_gather(x_flat[: B // (2 * K)])
    flat_b = sc_gather(x_flat[B // (2 * K):])

    rows_per_ch = CH * HIST // 2
    half_grid = BATCH // (2 * CH)
    wide_a = flat_a.reshape(hb // 2, 2 * D_MODEL)
    wide_b = flat_b.reshape(hb // 2, 2 * D_MODEL)
    part = pl.pallas_call(
        _format_body,
        grid=(half_grid,),
        in_specs=[pl.BlockSpec((rows_per_ch, 2 * D_MODEL), lambda i: (i, 0))],
        out_specs=pl.BlockSpec((HIST, D_MODEL, CH), lambda i: (0, 0, i)),
        out_shape=jax.ShapeDtypeStruct((HIST, D_MODEL, BATCH), jnp.float32),
    )(wide_a)
    tbd = pl.pallas_call(
        _format_body_alias,
        grid=(half_grid,),
        in_specs=[
            pl.BlockSpec(memory_space=pl.ANY),
            pl.BlockSpec((rows_per_ch, 2 * D_MODEL), lambda i: (i, 0)),
        ],
        out_specs=pl.BlockSpec(
            (HIST, D_MODEL, CH), lambda i: (0, 0, i + half_grid)
        ),
        out_shape=jax.ShapeDtypeStruct((HIST, D_MODEL, BATCH), jnp.float32),
        input_output_aliases={0: 0},
    )(part, wide_b)
    return jnp.transpose(tbd, (2, 0, 1))


# speedup vs baseline: 1.1641x; 1.0118x over previous
import functools

import jax
import jax.numpy as jnp
from jax import lax
from jax.experimental import pallas as pl
from jax.experimental.pallas import tpu as pltpu
from jax.experimental.pallas import tpu_sc as plsc

C_IN = 100000
D_MODEL = 64
BATCH = 4096
HIST = 50

NC = 2
NS = 16
NW = NC * NS

B = BATCH * HIST
K = 128
B_PER_W = B // NW
STEPS = B_PER_W // K
NBUF = 5
LA = 3
INNER = NBUF

CH = 512


def _gather_body(x_hbm, table_hbm, out_hbm, idx_v, rows, gsems, osems, *, steps):
    wid = lax.axis_index("s") * NC + lax.axis_index("c")
    base = wid * steps * K
    pltpu.sync_copy(x_hbm.at[pl.ds(wid * steps, steps)], idx_v)

    def gather(j, b):
        return pltpu.make_async_copy(
            table_hbm.at[idx_v.at[j]], rows[b], gsems[b]
        )

    def writeback(j, b):
        return pltpu.make_async_copy(
            rows[b], out_hbm.at[pl.ds(base + j * K, K)], osems[b]
        )

    for j in range(LA):
        gather(j, j).start()

    @pl.loop(0, steps // INNER)
    def _outer(p):
        j0 = p * INNER
        for t in range(INNER):
            j = j0 + t
            b = t % NBUF
            nb = (t + LA) % NBUF
            if t >= NBUF - LA:
                writeback(j + LA - NBUF, nb).wait()
            else:
                @pl.when(p > 0)
                def _():
                    writeback(j + LA - NBUF, nb).wait()

            @pl.when(j + LA < steps)
            def _():
                gather(j + LA, nb).start()

            gather(j, b).wait()
            writeback(j, b).start()

    for t in range(NBUF - LA):
        jt = steps - (NBUF - LA) + t
        writeback(jt, jt % NBUF).wait()


CP = 102400
HALF2 = CP // 2
CHW = 3200


def _repack_body(a_ref, o_ref):
    i = pl.program_id(0)
    o_ref[:, :D_MODEL] = a_ref[:, pl.ds(i * CHW, CHW)].T
    o_ref[:, D_MODEL:] = a_ref[:, pl.ds(HALF2 + i * CHW, CHW)].T


def _format_body_alias(dest_ref, src_ref, out_ref):
    del dest_ref
    _format_core(src_ref, out_ref)


def _format_body(src_ref, out_ref):
    _format_core(src_ref, out_ref)


def _format_core(src_ref, out_ref):
    m = src_ref[...].reshape(CH, HIST * D_MODEL)
    out_ref[...] = m.T.reshape(HIST, D_MODEL, CH)


@jax.jit
def kernel(x, W):
    x_t = jnp.where(x < HALF2, 2 * x, 2 * (x - HALF2) + 1)
    x_flat = x_t.reshape(B // K, K)
    w_t = jnp.pad(W, ((0, CP - C_IN), (0, 0))).T
    w_pairs = pl.pallas_call(
        _repack_body,
        grid=(HALF2 // CHW,),
        in_specs=[pl.BlockSpec((D_MODEL, CP), lambda i: (0, 0))],
        out_specs=pl.BlockSpec((CHW, 2 * D_MODEL), lambda i: (i, 0)),
        out_shape=jax.ShapeDtypeStruct((HALF2, 2 * D_MODEL), jnp.float32),
    )(w_t)
    w_lin = w_pairs.reshape(CP, D_MODEL)
    mesh = plsc.VectorSubcoreMesh(core_axis_name="c", subcore_axis_name="s")

    hsteps = STEPS // 2
    hb = B // 2

    def sc_gather(x_half):
        return pl.kernel(
            functools.partial(_gather_body, steps=hsteps),
            out_type=jax.ShapeDtypeStruct((hb, D_MODEL), jnp.float32),
            mesh=mesh,
            scratch_types=[
                pltpu.VMEM((hsteps, K), jnp.int32),
                tuple(pltpu.VMEM((K, D_MODEL), jnp.float32) for _ in range(NBUF)),
                tuple(pltpu.SemaphoreType.DMA for _ in range(NBUF)),
                tuple(pltpu.SemaphoreType.DMA for _ in range(NBUF)),
            ],
            compiler_params=pltpu.CompilerParams(use_tc_tiling_on_sc=False),
        )(x_half, w_lin)

    flat_a = sc_gather(x_flat[: B // (2 * K)])
    flat_b = sc_gather(x_flat[B // (2 * K):])

    rows_per_ch = CH * HIST // 2
    half_grid = BATCH // (2 * CH)
    wide_a = flat_a.reshape(hb // 2, 2 * D_MODEL)
    wide_b = flat_b.reshape(hb // 2, 2 * D_MODEL)
    part = pl.pallas_call(
        _format_body,
        grid=(half_grid,),
        in_specs=[pl.BlockSpec((rows_per_ch, 2 * D_MODEL), lambda i: (i, 0))],
        out_specs=pl.BlockSpec((HIST, D_MODEL, CH), lambda i: (0, 0, i)),
        out_shape=jax.ShapeDtypeStruct((HIST, D_MODEL, BATCH), jnp.float32),
    )(wide_a)
    tbd = pl.pallas_call(
        _format_body_alias,
        grid=(half_grid,),
        in_specs=[
            pl.BlockSpec(memory_space=pl.ANY),
            pl.BlockSpec((rows_per_ch, 2 * D_MODEL), lambda i: (i, 0)),
        ],
        out_specs=pl.BlockSpec(
            (HIST, D_MODEL, CH), lambda i: (0, 0, i + half_grid)
        ),
        out_shape=jax.ShapeDtypeStruct((HIST, D_MODEL, BATCH), jnp.float32),
        input_output_aliases={0: 0},
    )(part, wide_b)
    return jnp.transpose(tbd, (2, 0, 1))
